# Initial kernel scaffold; baseline (speedup 1.0000x reference)
import jax
import jax.numpy as jnp
from jax.experimental import pallas as pl


def kernel(x, table):
    raise NotImplementedError("write your pallas kernel here")


# speedup vs baseline: 1.7830x; 1.7830x over previous
import functools
import math

import jax
import jax.numpy as jnp
import numpy as np
from jax import lax
from jax.experimental import pallas as pl
from jax.experimental.pallas import tpu as pltpu
from jax.experimental.pallas import tpu_sc as plsc

_VSZ = 100000
_DSZ = 128
_MXLEN = 1000
_MAX_TIMESCALE = 10000.0
_B = 1024
_L = 200
_SCALE = math.sqrt(_DSZ)

_NC = 2
_NS = 16
_NW = _NC * _NS
_CHUNK = 128
_NROWS = _B * _L
_NCHUNKS = _NROWS // _CHUNK
_CPW = _NCHUNKS // _NW


def _pe_doubled():
    log_inc = math.log(_MAX_TIMESCALE) / _DSZ
    inv = np.exp(np.arange(0, _DSZ, 2).astype(np.float32) * -log_inc)
    pos = np.arange(0, _MXLEN).astype(np.float32)[:, None]
    pe = np.zeros((_MXLEN, _DSZ), dtype=np.float32)
    pe[:, 0::2] = np.sin(pos * inv)
    pe[:, 1::2] = np.cos(pos * inv)
    pe = pe[:_L]
    return np.concatenate([pe, pe], axis=0)


_MESH = plsc.VectorSubcoreMesh(core_axis_name="c", subcore_axis_name="s")


@functools.partial(
    pl.kernel,
    out_type=jax.ShapeDtypeStruct((_NROWS, _DSZ), jnp.float32),
    mesh=_MESH,
    scratch_types=[
        pltpu.VMEM((_CHUNK,), jnp.int32),
        pltpu.VMEM((_CHUNK, _DSZ), jnp.float32),
        pltpu.VMEM((2 * _L, _DSZ), jnp.float32),
        pltpu.SemaphoreType.DMA,
        pltpu.SemaphoreType.DMA,
    ],
)
def _emb_call(x_hbm, table_hbm, pe_hbm, out_hbm, idx_v, buf_v, pe_v, gsem, ssem):
    wid = lax.axis_index("s") * _NC + lax.axis_index("c")
    pltpu.sync_copy(pe_hbm, pe_v)

    def step(t, carry):
        c = wid * _CPW + t
        base = c * _CHUNK
        pltpu.sync_copy(x_hbm.at[pl.ds(base, _CHUNK)], idx_v)
        pltpu.async_copy(table_hbm.at[idx_v], buf_v, gsem).wait()
        p0 = lax.rem(base, _L)

        def row(r, rcarry):
            pr = p0 + r
            for d in range(_DSZ // 16):
                sl = pl.ds(d * 16, 16)
                buf_v[r, sl] = buf_v[r, sl] * _SCALE + pe_v[pr, sl]
            return rcarry

        lax.fori_loop(0, _CHUNK, row, 0, unroll=False)
        pltpu.async_copy(buf_v, out_hbm.at[pl.ds(base, _CHUNK)], ssem).wait()
        return carry

    lax.fori_loop(0, _CPW, step, 0, unroll=False)


def kernel(x, table):
    x_flat = x.reshape(_NROWS)
    pe2 = jnp.asarray(_pe_doubled())
    out = _emb_call(x_flat, table, pe2)
    return out.reshape(_B, _L, _DSZ)


# speedup vs baseline: 2.3185x; 1.3004x over previous
import functools
import math

import jax
import jax.numpy as jnp
import numpy as np
from jax import lax
from jax.experimental import pallas as pl
from jax.experimental.pallas import tpu as pltpu
from jax.experimental.pallas import tpu_sc as plsc

_VSZ = 100000
_DSZ = 128
_MXLEN = 1000
_MAX_TIMESCALE = 10000.0
_B = 1024
_L = 200
_SCALE = math.sqrt(_DSZ)

_NC = 2
_NS = 16
_NW = _NC * _NS
_CHUNK = 128
_NROWS = _B * _L
_NCHUNKS = _NROWS // _CHUNK
_CPW = _NCHUNKS // _NW


def _pe_doubled():
    log_inc = math.log(_MAX_TIMESCALE) / _DSZ
    inv = np.exp(np.arange(0, _DSZ, 2).astype(np.float32) * -log_inc)
    pos = np.arange(0, _MXLEN).astype(np.float32)[:, None]
    pe = np.zeros((_MXLEN, _DSZ), dtype=np.float32)
    pe[:, 0::2] = np.sin(pos * inv)
    pe[:, 1::2] = np.cos(pos * inv)
    pe = pe[:_L]
    return np.concatenate([pe, pe], axis=0)


_MESH = plsc.VectorSubcoreMesh(core_axis_name="c", subcore_axis_name="s")


@functools.partial(
    pl.kernel,
    out_type=jax.ShapeDtypeStruct((_NROWS, _DSZ), jnp.float32),
    mesh=_MESH,
    scratch_types=[
        pltpu.VMEM((_CPW * _CHUNK,), jnp.int32),
        pltpu.VMEM((_CHUNK, _DSZ), jnp.float32),
        pltpu.VMEM((_CHUNK, _DSZ), jnp.float32),
        pltpu.VMEM((2 * _L, _DSZ), jnp.float32),
        pltpu.SemaphoreType.DMA,
        pltpu.SemaphoreType.DMA,
        pltpu.SemaphoreType.DMA,
        pltpu.SemaphoreType.DMA,
    ],
)
def _emb_call(x_hbm, table_hbm, pe_hbm, out_hbm,
              idx_v, buf0, buf1, pe_v, gsem0, gsem1, ssem0, ssem1):
    wid = lax.axis_index("s") * _NC + lax.axis_index("c")
    c0 = wid * _CPW
    pltpu.sync_copy(x_hbm.at[pl.ds(c0 * _CHUNK, _CPW * _CHUNK)], idx_v)
    pltpu.sync_copy(pe_hbm, pe_v)

    bufs = (buf0, buf1)
    gsems = (gsem0, gsem1)
    ssems = (ssem0, ssem1)

    def gather(t, slot):
        pltpu.async_copy(
            table_hbm.at[idx_v.at[pl.ds(t * _CHUNK, _CHUNK)]],
            bufs[slot], gsems[slot])

    def compute_store(t, slot):
        buf = bufs[slot]
        base = (c0 + t) * _CHUNK
        p0 = lax.rem(base, _L)

        def row(r, rcarry):
            pr = p0 + r
            for d in range(_DSZ // 16):
                sl = pl.ds(d * 16, 16)
                buf[r, sl] = buf[r, sl] * _SCALE + pe_v[pr, sl]
            return rcarry

        lax.fori_loop(0, _CHUNK, row, 0, unroll=2)
        pltpu.async_copy(buf, out_hbm.at[pl.ds(base, _CHUNK)], ssems[slot])

    gather(0, 0)
    pltpu.make_async_copy(
        table_hbm.at[idx_v.at[pl.ds(0, _CHUNK)]], buf0, gsem0).wait()
    gather(1, 1)
    compute_store(0, 0)

    def pair(i, carry):
        for b, slot in ((1, 1), (2, 0)):
            t = 2 * i + b
            pltpu.make_async_copy(
                table_hbm.at[idx_v.at[pl.ds(t * _CHUNK, _CHUNK)]],
                bufs[slot], gsems[slot]).wait()
            other = 1 - slot
            pltpu.make_async_copy(
                bufs[other], out_hbm.at[pl.ds(0, _CHUNK)], ssems[other]).wait()
            gather(t + 1, other)
            compute_store(t, slot)
        return carry

    lax.fori_loop(0, (_CPW - 2) // 2, pair, 0, unroll=False)

    pltpu.make_async_copy(
        table_hbm.at[idx_v.at[pl.ds((_CPW - 1) * _CHUNK, _CHUNK)]],
        buf1, gsem1).wait()
    compute_store(_CPW - 1, 1)
    pltpu.make_async_copy(buf0, out_hbm.at[pl.ds(0, _CHUNK)], ssem0).wait()
    pltpu.make_async_copy(buf1, out_hbm.at[pl.ds(0, _CHUNK)], ssem1).wait()


def kernel(x, table):
    x_flat = x.reshape(_NROWS)
    pe2 = jnp.asarray(_pe_doubled())
    out = _emb_call(x_flat, table, pe2)
    return out.reshape(_B, _L, _DSZ)


# speedup vs baseline: 6.0781x; 2.6215x over previous
import functools
import math

import jax
import jax.numpy as jnp
import numpy as np
from jax import lax
from jax.experimental import pallas as pl
from jax.experimental.pallas import tpu as pltpu
from jax.experimental.pallas import tpu_sc as plsc

_VSZ = 100000
_DSZ = 128
_MXLEN = 1000
_MAX_TIMESCALE = 10000.0
_B = 1024
_L = 200
_SCALE = math.sqrt(_DSZ)

_NC = 2
_NS = 16
_NW = _NC * _NS
_CHUNK = 128
_NROWS = _B * _L
_NCHUNKS = _NROWS // _CHUNK
_CPW = _NCHUNKS // _NW


def _pe_doubled():
    log_inc = math.log(_MAX_TIMESCALE) / _DSZ
    inv = np.exp(np.arange(0, _DSZ, 2).astype(np.float32) * -log_inc)
    pos = np.arange(0, _MXLEN).astype(np.float32)[:, None]
    pe = np.zeros((_MXLEN, _DSZ), dtype=np.float32)
    pe[:, 0::2] = np.sin(pos * inv)
    pe[:, 1::2] = np.cos(pos * inv)
    pe = pe[:_L]
    return np.concatenate([pe, pe], axis=0)


_MESH = plsc.VectorSubcoreMesh(core_axis_name="c", subcore_axis_name="s")


@functools.partial(
    pl.kernel,
    out_type=jax.ShapeDtypeStruct((_NROWS, _DSZ), jnp.float32),
    mesh=_MESH,
    scratch_types=[
        pltpu.VMEM((_CPW * _CHUNK,), jnp.int32),
        pltpu.VMEM((_CHUNK, _DSZ), jnp.float32),
        pltpu.VMEM((_CHUNK, _DSZ), jnp.float32),
        pltpu.VMEM((2 * _L, _DSZ), jnp.float32),
        pltpu.SemaphoreType.DMA,
        pltpu.SemaphoreType.DMA,
        pltpu.SemaphoreType.DMA,
        pltpu.SemaphoreType.DMA,
    ],
)
def _emb_call(x_hbm, table_hbm, pe_hbm, out_hbm,
              idx_v, buf0, buf1, pe_v, gsem0, gsem1, ssem0, ssem1):
    wid = lax.axis_index("s") * _NC + lax.axis_index("c")
    c0 = wid * _CPW
    pltpu.sync_copy(x_hbm.at[pl.ds(c0 * _CHUNK, _CPW * _CHUNK)], idx_v)
    pltpu.sync_copy(pe_hbm, pe_v)

    bufs = (buf0, buf1)
    gsems = (gsem0, gsem1)
    ssems = (ssem0, ssem1)

    def gather(t, slot):
        pltpu.async_copy(
            table_hbm.at[idx_v.at[pl.ds(t * _CHUNK, _CHUNK)]],
            bufs[slot], gsems[slot])

    def compute_store(t, slot):
        buf = bufs[slot]
        base = (c0 + t) * _CHUNK
        p0 = lax.rem(base, _L)

        @plsc.parallel_loop(0, _CHUNK, step=1, unroll=4)
        def row(r):
            pr = p0 + r
            for d in range(_DSZ // 16):
                sl = pl.ds(d * 16, 16)
                buf[r, sl] = buf[r, sl] * _SCALE + pe_v[pr, sl]
        pltpu.async_copy(buf, out_hbm.at[pl.ds(base, _CHUNK)], ssems[slot])

    gather(0, 0)
    pltpu.make_async_copy(
        table_hbm.at[idx_v.at[pl.ds(0, _CHUNK)]], buf0, gsem0).wait()
    gather(1, 1)
    compute_store(0, 0)

    def pair(i, carry):
        for b, slot in ((1, 1), (2, 0)):
            t = 2 * i + b
            pltpu.make_async_copy(
                table_hbm.at[idx_v.at[pl.ds(t * _CHUNK, _CHUNK)]],
                bufs[slot], gsems[slot]).wait()
            other = 1 - slot
            pltpu.make_async_copy(
                bufs[other], out_hbm.at[pl.ds(0, _CHUNK)], ssems[other]).wait()
            gather(t + 1, other)
            compute_store(t, slot)
        return carry

    lax.fori_loop(0, (_CPW - 2) // 2, pair, 0, unroll=False)

    pltpu.make_async_copy(
        table_hbm.at[idx_v.at[pl.ds((_CPW - 1) * _CHUNK, _CHUNK)]],
        buf1, gsem1).wait()
    compute_store(_CPW - 1, 1)
    pltpu.make_async_copy(buf0, out_hbm.at[pl.ds(0, _CHUNK)], ssem0).wait()
    pltpu.make_async_copy(buf1, out_hbm.at[pl.ds(0, _CHUNK)], ssem1).wait()


def kernel(x, table):
    x_flat = x.reshape(_NROWS)
    pe2 = jnp.asarray(_pe_doubled())
    out = _emb_call(x_flat, table, pe2)
    return out.reshape(_B, _L, _DSZ)


# speedup vs baseline: 7.1561x; 1.1774x over previous
import functools
import math

import jax
import jax.numpy as jnp
import numpy as np
from jax import lax
from jax.experimental import pallas as pl
from jax.experimental.pallas import tpu as pltpu
from jax.experimental.pallas import tpu_sc as plsc

_VSZ = 100000
_DSZ = 128
_MXLEN = 1000
_MAX_TIMESCALE = 10000.0
_B = 1024
_L = 200
_SCALE = math.sqrt(_DSZ)

_NC = 2
_NS = 16
_NW = _NC * _NS
_CHUNK = 128
_NROWS = _B * _L
_NCHUNKS = _NROWS // _CHUNK
_CPW = _NCHUNKS // _NW
_NBUF = 4
_PEROWS = _L + _CHUNK


def _pe_tiled():
    log_inc = math.log(_MAX_TIMESCALE) / _DSZ
    inv = np.exp(np.arange(0, _DSZ, 2).astype(np.float32) * -log_inc)
    pos = np.arange(0, _MXLEN).astype(np.float32)[:, None]
    pe = np.zeros((_MXLEN, _DSZ), dtype=np.float32)
    pe[:, 0::2] = np.sin(pos * inv)
    pe[:, 1::2] = np.cos(pos * inv)
    pe = pe[:_L]
    return np.concatenate([pe, pe[: _PEROWS - _L]], axis=0)


_MESH = plsc.VectorSubcoreMesh(core_axis_name="c", subcore_axis_name="s")


@functools.partial(
    pl.kernel,
    out_type=jax.ShapeDtypeStruct((_NROWS, _DSZ), jnp.float32),
    mesh=_MESH,
    scratch_types=[
        pltpu.VMEM((_CPW * _CHUNK,), jnp.int32),
        *[pltpu.VMEM((_CHUNK, _DSZ), jnp.float32) for _ in range(_NBUF)],
        pltpu.VMEM((_PEROWS, _DSZ), jnp.float32),
        pltpu.SemaphoreType.DMA,
        *[pltpu.SemaphoreType.DMA for _ in range(_NBUF)],
        *[pltpu.SemaphoreType.DMA for _ in range(_NBUF)],
    ],
)
def _emb_call(x_hbm, table_hbm, pe_hbm, out_hbm,
              idx_v, b0, b1, b2, b3, pe_v, psem,
              g0, g1, g2, g3, s0, s1, s2, s3):
    wid = lax.axis_index("s") * _NC + lax.axis_index("c")
    c0 = wid * _CPW

    bufs = (b0, b1, b2, b3)
    gsems = (g0, g1, g2, g3)
    ssems = (s0, s1, s2, s3)

    pltpu.sync_copy(x_hbm.at[pl.ds(c0 * _CHUNK, _CPW * _CHUNK)], idx_v)
    pltpu.async_copy(pe_hbm, pe_v, psem)

    def gather(t, slot):
        pltpu.async_copy(
            table_hbm.at[idx_v.at[pl.ds(t * _CHUNK, _CHUNK)]],
            bufs[slot], gsems[slot])

    def wait_gather(t, slot):
        pltpu.make_async_copy(
            table_hbm.at[idx_v.at[pl.ds(t * _CHUNK, _CHUNK)]],
            bufs[slot], gsems[slot]).wait()

    def wait_store(slot):
        pltpu.make_async_copy(
            bufs[slot], out_hbm.at[pl.ds(0, _CHUNK)], ssems[slot]).wait()

    def compute_store(t, slot):
        buf = bufs[slot]
        base = (c0 + t) * _CHUNK
        p0 = lax.rem(base, _L)

        @plsc.parallel_loop(0, _CHUNK, step=1, unroll=4)
        def row(r):
            pr = p0 + r
            for d in range(_DSZ // 16):
                sl = pl.ds(d * 16, 16)
                buf[r, sl] = buf[r, sl] * _SCALE + pe_v[pr, sl]

        pltpu.async_copy(buf, out_hbm.at[pl.ds(base, _CHUNK)], ssems[slot])

    for t in range(_NBUF - 1):
        gather(t, t)
    pltpu.make_async_copy(pe_hbm, pe_v, psem).wait()

    wait_gather(0, 0)
    compute_store(0, 0)
    gather(_NBUF - 1, _NBUF - 1)

    def group(i, carry):
        for b in range(_NBUF):
            t = _NBUF * i + 1 + b
            slot = (1 + b) % _NBUF
            wait_gather(t, slot)
            compute_store(t, slot)
            s3 = b
            wait_store(s3)
            gather(t + _NBUF - 1, s3)
        return carry

    lax.fori_loop(0, (_CPW - 2 - (_NBUF - 1)) // _NBUF, group, 0,
                  unroll=False)

    for t in range(_CPW - _NBUF - 1, _CPW):
        slot = t % _NBUF
        wait_gather(t, slot)
        compute_store(t, slot)
        if t + _NBUF - 1 < _CPW:
            s3 = (t + _NBUF - 1) % _NBUF
            wait_store(s3)
            gather(t + _NBUF - 1, s3)

    for slot in range(_NBUF):
        wait_store(slot)


def kernel(x, table):
    x_flat = x.reshape(_NROWS)
    pe = jnp.asarray(_pe_tiled())
    out = _emb_call(x_flat, table, pe)
    return out.reshape(_B, _L, _DSZ)


# speedup vs baseline: 7.6027x; 1.0624x over previous
import functools
import math

import jax
import jax.numpy as jnp
import numpy as np
from jax import lax
from jax.experimental import pallas as pl
from jax.experimental.pallas import tpu as pltpu
from jax.experimental.pallas import tpu_sc as plsc

_VSZ = 100000
_DSZ = 128
_MXLEN = 1000
_MAX_TIMESCALE = 10000.0
_B = 1024
_L = 200
_SCALE = math.sqrt(_DSZ)

_NC = 2
_NS = 16
_NW = _NC * _NS
_CHUNK = 128
_NROWS = _B * _L
_NBC = _B // _CHUNK
_LPW = _L // (_NW // _NBC)
_CPW = _LPW
_NBUF = 4


def _pe_table():
    log_inc = math.log(_MAX_TIMESCALE) / _DSZ
    inv = np.exp(np.arange(0, _DSZ, 2).astype(np.float32) * -log_inc)
    pos = np.arange(0, _MXLEN).astype(np.float32)[:, None]
    pe = np.zeros((_MXLEN, _DSZ), dtype=np.float32)
    pe[:, 0::2] = np.sin(pos * inv)
    pe[:, 1::2] = np.cos(pos * inv)
    return pe[:_L]


_MESH = plsc.VectorSubcoreMesh(core_axis_name="c", subcore_axis_name="s")


@functools.partial(
    pl.kernel,
    out_type=jax.ShapeDtypeStruct((_NROWS, _DSZ), jnp.float32),
    mesh=_MESH,
    scratch_types=[
        pltpu.VMEM((_L, _CHUNK), jnp.int32),
        *[pltpu.VMEM((_CHUNK, _DSZ), jnp.float32) for _ in range(_NBUF)],
        pltpu.VMEM((_L, _DSZ), jnp.float32),
        *[pltpu.VMEM((1, _CHUNK), jnp.int32) for _ in range(_NBUF)],
        pltpu.SemaphoreType.DMA,
        *[pltpu.SemaphoreType.DMA for _ in range(_NBUF)],
        *[pltpu.SemaphoreType.DMA for _ in range(_NBUF)],
    ],
)
def _emb_call(xt_hbm, table_hbm, pe_hbm, out_hbm,
              idx_v, b0, b1, b2, b3, pe_v, o0, o1, o2, o3, psem,
              g0, g1, g2, g3, s0, s1, s2, s3):
    wid = lax.axis_index("s") * _NC + lax.axis_index("c")
    bc = lax.div(wid, _NW // _NBC)
    l0 = lax.rem(wid, _NW // _NBC) * _LPW
    obase0 = bc * (_CHUNK * _L) + l0

    bufs = (b0, b1, b2, b3)
    oidx = (o0, o1, o2, o3)
    gsems = (g0, g1, g2, g3)
    ssems = (s0, s1, s2, s3)

    pltpu.sync_copy(xt_hbm.at[:, pl.ds(bc * _CHUNK, _CHUNK)], idx_v)
    pltpu.async_copy(pe_hbm, pe_v, psem)

    def gather(t, slot):
        pltpu.async_copy(
            table_hbm.at[idx_v.at[l0 + t]], bufs[slot], gsems[slot])

    def wait_gather(t, slot):
        pltpu.make_async_copy(
            table_hbm.at[idx_v.at[l0 + t]], bufs[slot], gsems[slot]).wait()

    def wait_store(slot):
        pltpu.make_async_copy(
            bufs[slot], out_hbm.at[oidx[slot].at[0]], ssems[slot]).wait()

    def compute_store(t, slot):
        buf = bufs[slot]
        l = l0 + t
        pev = [pe_v[l, pl.ds(d * 16, 16)] for d in range(_DSZ // 16)]

        @plsc.parallel_loop(0, _CHUNK, step=1, unroll=4)
        def row(r):
            for d in range(_DSZ // 16):
                sl = pl.ds(d * 16, 16)
                buf[r, sl] = buf[r, sl] * _SCALE + pev[d]

        base = obase0 + t
        for j in range(_CHUNK // 16):
            sl = pl.ds(j * 16, 16)
            oidx[slot][0, sl] = (
                lax.iota(jnp.int32, 16) + (j * 16)) * _L + base
        pltpu.async_copy(buf, out_hbm.at[oidx[slot].at[0]], ssems[slot])

    for t in range(_NBUF - 1):
        gather(t, t)
    pltpu.make_async_copy(pe_hbm, pe_v, psem).wait()

    wait_gather(0, 0)
    compute_store(0, 0)
    gather(_NBUF - 1, _NBUF - 1)

    def group(i, carry):
        for b in range(_NBUF):
            t = _NBUF * i + 1 + b
            slot = (1 + b) % _NBUF
            wait_gather(t, slot)
            compute_store(t, slot)
            s3 = b
            wait_store(s3)
            gather(t + _NBUF - 1, s3)
        return carry

    lax.fori_loop(0, (_CPW - 2 - (_NBUF - 1)) // _NBUF, group, 0,
                  unroll=False)

    for t in range(_CPW - _NBUF - 1, _CPW):
        slot = t % _NBUF
        wait_gather(t, slot)
        compute_store(t, slot)
        if t + _NBUF - 1 < _CPW:
            s3 = (t + _NBUF - 1) % _NBUF
            wait_store(s3)
            gather(t + _NBUF - 1, s3)

    for slot in range(_NBUF):
        wait_store(slot)


def kernel(x, table):
    xt = x.T
    pe = jnp.asarray(_pe_table())
    out = _emb_call(xt, table, pe)
    return out.reshape(_B, _L, _DSZ)


# speedup vs baseline: 7.6367x; 1.0045x over previous
import functools
import math

import jax
import jax.numpy as jnp
import numpy as np
from jax import lax
from jax.experimental import pallas as pl
from jax.experimental.pallas import tpu as pltpu
from jax.experimental.pallas import tpu_sc as plsc

_VSZ = 100000
_DSZ = 128
_MXLEN = 1000
_MAX_TIMESCALE = 10000.0
_B = 1024
_L = 200
_SCALE = math.sqrt(_DSZ)

_NC = 2
_NS = 16
_NW = _NC * _NS
_CHUNK = 128
_NROWS = _B * _L
_NBC = _B // _CHUNK
_LPW = _L // (_NW // _NBC)
_CPW = _LPW
_NBUF = 5
_IDXROWS = 56


def _pe_table():
    log_inc = math.log(_MAX_TIMESCALE) / _DSZ
    inv = np.exp(np.arange(0, _DSZ, 2).astype(np.float32) * -log_inc)
    pos = np.arange(0, _MXLEN).astype(np.float32)[:, None]
    pe = np.zeros((_MXLEN, _DSZ), dtype=np.float32)
    pe[:, 0::2] = np.sin(pos * inv)
    pe[:, 1::2] = np.cos(pos * inv)
    return pe[:_L]


_MESH = plsc.VectorSubcoreMesh(core_axis_name="c", subcore_axis_name="s")


@functools.partial(
    pl.kernel,
    out_type=jax.ShapeDtypeStruct((_NROWS, _DSZ), jnp.float32),
    mesh=_MESH,
    scratch_types=[
        pltpu.VMEM((_IDXROWS, _CHUNK), jnp.int32),
        *[pltpu.VMEM((_CHUNK, _DSZ), jnp.float32) for _ in range(_NBUF)],
        pltpu.VMEM((_L, _DSZ), jnp.float32),
        *[pltpu.VMEM((1, _CHUNK), jnp.int32) for _ in range(_NBUF)],
        pltpu.SemaphoreType.DMA,
        *[pltpu.SemaphoreType.DMA for _ in range(_NBUF)],
        *[pltpu.SemaphoreType.DMA for _ in range(_NBUF)],
    ],
)
def _emb_call(xt_hbm, table_hbm, pe_hbm, out_hbm,
              idx_v, b0, b1, b2, b3, b4, pe_v, o0, o1, o2, o3, o4, psem,
              g0, g1, g2, g3, g4, s0, s1, s2, s3, s4):
    wid = lax.axis_index("s") * _NC + lax.axis_index("c")
    bc = lax.div(wid, _NW // _NBC)
    l0 = lax.rem(wid, _NW // _NBC) * _LPW
    lofs = lax.rem(l0, 8)
    obase0 = bc * (_CHUNK * _L) + l0

    bufs = (b0, b1, b2, b3, b4)
    oidx = (o0, o1, o2, o3, o4)
    gsems = (g0, g1, g2, g3, g4)
    ssems = (s0, s1, s2, s3, s4)

    lstart = pl.multiple_of(l0 - lofs, 8)
    pltpu.sync_copy(
        xt_hbm.at[pl.ds(lstart, _IDXROWS), pl.ds(bc * _CHUNK, _CHUNK)],
        idx_v)
    pltpu.async_copy(pe_hbm, pe_v, psem)

    def gather(t, slot):
        pltpu.async_copy(
            table_hbm.at[idx_v.at[lofs + t]], bufs[slot], gsems[slot])

    def wait_gather(t, slot):
        pltpu.make_async_copy(
            table_hbm.at[idx_v.at[lofs + t]], bufs[slot], gsems[slot]).wait()

    def wait_store(slot):
        pltpu.make_async_copy(
            bufs[slot], out_hbm.at[oidx[slot].at[0]], ssems[slot]).wait()

    def compute_store(t, slot):
        buf = bufs[slot]
        l = l0 + t
        pev = [pe_v[l, pl.ds(d * 16, 16)] for d in range(_DSZ // 16)]

        @plsc.parallel_loop(0, _CHUNK, step=1, unroll=4)
        def row(r):
            for d in range(_DSZ // 16):
                sl = pl.ds(d * 16, 16)
                buf[r, sl] = buf[r, sl] * _SCALE + pev[d]

        base = obase0 + t
        for j in range(_CHUNK // 16):
            sl = pl.ds(j * 16, 16)
            oidx[slot][0, sl] = (
                lax.iota(jnp.int32, 16) + (j * 16)) * _L + base
        pltpu.async_copy(buf, out_hbm.at[oidx[slot].at[0]], ssems[slot])

    for t in range(_NBUF - 1):
        gather(t, t)
    pltpu.make_async_copy(pe_hbm, pe_v, psem).wait()

    wait_gather(0, 0)
    compute_store(0, 0)
    gather(_NBUF - 1, _NBUF - 1)

    ngroups = (_CPW - 2 - (_NBUF - 1)) // _NBUF

    def group(i, carry):
        for b in range(_NBUF):
            t = _NBUF * i + 1 + b
            slot = (1 + b) % _NBUF
            wait_gather(t, slot)
            compute_store(t, slot)
            s3 = b
            wait_store(s3)
            gather(t + _NBUF - 1, s3)
        return carry

    lax.fori_loop(0, ngroups, group, 0, unroll=False)

    for t in range(1 + _NBUF * ngroups, _CPW):
        slot = t % _NBUF
        wait_gather(t, slot)
        compute_store(t, slot)
        if t + _NBUF - 1 < _CPW:
            s3 = (t + _NBUF - 1) % _NBUF
            wait_store(s3)
            gather(t + _NBUF - 1, s3)

    for slot in range(_NBUF):
        wait_store(slot)


def kernel(x, table):
    xt = x.T
    pe = jnp.asarray(_pe_table())
    out = _emb_call(xt, table, pe)
    return out.reshape(_B, _L, _DSZ)
